# Initial kernel scaffold; baseline (speedup 1.0000x reference)
import functools

import jax
import jax.numpy as jnp
from jax import lax
from jax.experimental import pallas as pl
from jax.experimental.pallas import tpu as pltpu
from jax.experimental.pallas import tpu_sc as plsc

N = 10000
E = 320000
D = 128
H = 64
NPAD = 10016
RPT = NPAD // 16
CH = 128
NCH = 160
EPT = NCH * CH
EPAD = 16 * EPT
DW = 8
PADROW = N

_MESH = plsc.VectorSubcoreMesh(core_axis_name="c", subcore_axis_name="s")


@functools.partial(
    pl.kernel,
    out_type=(jax.ShapeDtypeStruct((NPAD, DW), jnp.float32),
              jax.ShapeDtypeStruct((NPAD, DW), jnp.float32)),
    mesh=_MESH,
    scratch_types=[
        pltpu.VMEM_SHARED((NPAD, DW), jnp.float32),
        pltpu.VMEM((NCH, CH), jnp.int32),
        pltpu.VMEM((CH, DW), jnp.float32),
    ],
)
def _sc_degrees(src_t, dst_t, zeros8, ones8, outdeg, indeg, acc, idx_v, ones_v):
    c = lax.axis_index("c")
    s = lax.axis_index("s")
    rs = pl.ds(s * RPT, RPT)
    pltpu.sync_copy(zeros8.at[rs], acc.at[rs])
    pltpu.sync_copy(ones8, ones_v)

    @pl.when(c == 0)
    def _():
        pltpu.sync_copy(src_t.at[s], idx_v)

    @pl.when(c == 1)
    def _():
        pltpu.sync_copy(dst_t.at[s], idx_v)

    plsc.subcore_barrier()

    @pl.loop(0, NCH)
    def _(j):
        pltpu.sync_copy(ones_v, acc.at[idx_v.at[j]], add=True)

    plsc.subcore_barrier()

    @pl.when(c == 0)
    def _():
        pltpu.sync_copy(acc.at[rs], outdeg.at[rs])

    @pl.when(c == 1)
    def _():
        pltpu.sync_copy(acc.at[rs], indeg.at[rs])


@functools.partial(
    pl.kernel,
    out_type=(jax.ShapeDtypeStruct((NPAD, H), jnp.float32),
              jax.ShapeDtypeStruct((NPAD, H), jnp.float32)),
    mesh=_MESH,
    scratch_types=[
        pltpu.VMEM_SHARED((NPAD, H), jnp.float32),
        pltpu.VMEM_SHARED((NPAD, H), jnp.float32),
        pltpu.VMEM((NCH, CH), jnp.int32),
        pltpu.VMEM((NCH, CH), jnp.int32),
        pltpu.VMEM((CH, H), jnp.float32),
        pltpu.SemaphoreType.DMA,
    ],
)
def _sc_gather_scatter(ha, hb, src_t, dst_t, zeros64, outa, outb,
                       table, acc, sidx, didx, rows, sem):
    c = lax.axis_index("c")
    s = lax.axis_index("s")
    rs = pl.ds(s * RPT, RPT)

    @pl.when(c == 0)
    def _():
        pltpu.sync_copy(ha.at[rs], table.at[rs])

    @pl.when(c == 1)
    def _():
        pltpu.sync_copy(hb.at[rs], table.at[rs])

    pltpu.sync_copy(zeros64.at[rs], acc.at[rs])
    pltpu.sync_copy(src_t.at[s], sidx)
    pltpu.sync_copy(dst_t.at[s], didx)
    plsc.subcore_barrier()

    @pl.loop(0, NCH)
    def _(j):
        pltpu.async_copy(table.at[sidx.at[j]], rows, sem).wait()
        pltpu.sync_copy(rows, acc.at[didx.at[j]], add=True)

    plsc.subcore_barrier()

    @pl.when(c == 0)
    def _():
        pltpu.sync_copy(acc.at[rs], outa.at[rs])

    @pl.when(c == 1)
    def _():
        pltpu.sync_copy(acc.at[rs], outb.at[rs])


_RB = 512
_GRID = (NPAD + _RB - 1) // _RB


def _tc_pre_body(h_ref, deg_ref, a_ref, b_ref):
    norm = lax.rsqrt(jnp.maximum(deg_ref[:, :1], 1.0))
    x = h_ref[...] * norm
    a_ref[...] = x[:, :H]
    b_ref[...] = x[:, H:]


def _tc_pre(h_pad, out_deg):
    return pl.pallas_call(
        _tc_pre_body,
        grid=(_GRID,),
        in_specs=[pl.BlockSpec((_RB, D), lambda i: (i, 0)),
                  pl.BlockSpec((_RB, DW), lambda i: (i, 0))],
        out_specs=[pl.BlockSpec((_RB, H), lambda i: (i, 0)),
                   pl.BlockSpec((_RB, H), lambda i: (i, 0))],
        out_shape=[jax.ShapeDtypeStruct((NPAD, H), jnp.float32)] * 2,
    )(h_pad, out_deg)


def _tc_mid_body(a_ref, b_ref, ind_ref, outd_ref, w_ref, bias_ref,
                 oa_ref, ob_ref):
    agg = jnp.concatenate([a_ref[...], b_ref[...]], axis=1)
    agg = agg * lax.rsqrt(jnp.maximum(ind_ref[:, :1], 1.0))
    y = jnp.dot(agg, w_ref[...], preferred_element_type=jnp.float32,
                precision=lax.Precision.HIGHEST)
    y = jnp.maximum(y + bias_ref[...], 0.0)
    y = y * lax.rsqrt(jnp.maximum(outd_ref[:, :1], 1.0))
    oa_ref[...] = y[:, :H]
    ob_ref[...] = y[:, H:]


def _tc_mid(acc_a, acc_b, in_deg, out_deg, W, bias):
    return pl.pallas_call(
        _tc_mid_body,
        grid=(_GRID,),
        in_specs=[pl.BlockSpec((_RB, H), lambda i: (i, 0)),
                  pl.BlockSpec((_RB, H), lambda i: (i, 0)),
                  pl.BlockSpec((_RB, DW), lambda i: (i, 0)),
                  pl.BlockSpec((_RB, DW), lambda i: (i, 0)),
                  pl.BlockSpec((D, D), lambda i: (0, 0)),
                  pl.BlockSpec((1, D), lambda i: (0, 0))],
        out_specs=[pl.BlockSpec((_RB, H), lambda i: (i, 0)),
                   pl.BlockSpec((_RB, H), lambda i: (i, 0))],
        out_shape=[jax.ShapeDtypeStruct((NPAD, H), jnp.float32)] * 2,
    )(acc_a, acc_b, in_deg, out_deg, W, bias)


def _tc_last_body(a_ref, b_ref, ind_ref, w_ref, bias_ref, o_ref):
    agg = jnp.concatenate([a_ref[...], b_ref[...]], axis=1)
    agg = agg * lax.rsqrt(jnp.maximum(ind_ref[:, :1], 1.0))
    o_ref[...] = jnp.dot(agg, w_ref[...], preferred_element_type=jnp.float32,
                         precision=lax.Precision.HIGHEST) + bias_ref[...]


def _tc_last(acc_a, acc_b, in_deg, W, bias):
    return pl.pallas_call(
        _tc_last_body,
        grid=(_GRID,),
        in_specs=[pl.BlockSpec((_RB, H), lambda i: (i, 0)),
                  pl.BlockSpec((_RB, H), lambda i: (i, 0)),
                  pl.BlockSpec((_RB, DW), lambda i: (i, 0)),
                  pl.BlockSpec((D, D), lambda i: (0, 0)),
                  pl.BlockSpec((1, D), lambda i: (0, 0))],
        out_specs=pl.BlockSpec((_RB, D), lambda i: (i, 0)),
        out_shape=jax.ShapeDtypeStruct((N, D), jnp.float32),
    )(acc_a, acc_b, in_deg, W, bias)


def kernel(h, e, edge_index, W0, b0, W1, b1, W2, b2):
    src = edge_index[0]
    dst = edge_index[1]
    pad_idx = jnp.full((EPAD - E,), PADROW, dtype=jnp.int32)
    src_t = jnp.concatenate([src, pad_idx]).reshape(16, NCH, CH)
    dst_t = jnp.concatenate([dst, pad_idx]).reshape(16, NCH, CH)
    h_pad = jnp.pad(h, ((0, NPAD - N), (0, 0)))
    zeros8 = jnp.zeros((NPAD, DW), jnp.float32)
    ones8 = jnp.ones((CH, DW), jnp.float32)
    zeros64 = jnp.zeros((NPAD, H), jnp.float32)

    out_deg, in_deg = _sc_degrees(src_t, dst_t, zeros8, ones8)
    ha, hb = _tc_pre(h_pad, out_deg)

    out = None
    for (W, b, last) in ((W0, b0, False), (W1, b1, False), (W2, b2, True)):
        acc_a, acc_b = _sc_gather_scatter(ha, hb, src_t, dst_t, zeros64)
        if last:
            out = _tc_last(acc_a, acc_b, in_deg, W, b.reshape(1, D))
        else:
            ha, hb = _tc_mid(acc_a, acc_b, in_deg, out_deg, W, b.reshape(1, D))
    return (out, e)


# speedup vs baseline: 1.2818x; 1.2818x over previous
import functools

import jax
import jax.numpy as jnp
from jax import lax
from jax.experimental import pallas as pl
from jax.experimental.pallas import tpu as pltpu
from jax.experimental.pallas import tpu_sc as plsc

N = 10000
E = 320000
D = 128
NH = 5120
NPAD = 2 * NH
ACCR = 5248
TRASH = 5184
ZRPT = ACCR // 16
ORPT = NH // 16
CH = 128
NCH = 160
EPT = NCH * CH
EPAD = 16 * EPT
PADROW = N

_MESH = plsc.VectorSubcoreMesh(core_axis_name="c", subcore_axis_name="s")


@functools.partial(
    pl.kernel,
    out_type=jax.ShapeDtypeStruct((NPAD, D), jnp.float32),
    mesh=_MESH,
    scratch_types=[
        pltpu.VMEM_SHARED((ACCR, D), jnp.float32),
        pltpu.VMEM((NCH, CH), jnp.int32),
        pltpu.VMEM((NCH, CH), jnp.int32),
        pltpu.VMEM((CH,), jnp.int32),
        pltpu.VMEM((CH,), jnp.int32),
        pltpu.VMEM((CH, D), jnp.float32),
        pltpu.SemaphoreType.DMA,
    ],
)
def _sc_pass(table_h, si, di, zeros, out,
             acc, sidx, didx, sidx1, didx1, rows, sem):
    c = lax.axis_index("c")
    s = lax.axis_index("s")
    base = c * NH

    pltpu.sync_copy(zeros.at[pl.ds(s * ZRPT, ZRPT)],
                    acc.at[pl.ds(s * ZRPT, ZRPT)])
    pltpu.sync_copy(si.at[s], sidx)
    pltpu.sync_copy(di.at[s], didx)
    plsc.subcore_barrier()

    @pl.loop(0, NCH)
    def _(j):
        for k in range(CH // 16):
            sl = pl.ds(k * 16, 16)
            sidx1[sl] = sidx[j, sl]
            v = didx[j, sl] - base
            ok = (v >= 0) & (v < NH)
            didx1[sl] = jnp.where(ok, v, TRASH)
        pltpu.async_copy(table_h.at[sidx1], rows, sem).wait()
        pltpu.sync_copy(rows, acc.at[didx1], add=True)

    plsc.subcore_barrier()
    pltpu.sync_copy(acc.at[pl.ds(s * ORPT, ORPT)],
                    out.at[pl.ds(base + s * ORPT, ORPT)])


_RB = 512
_GRID = NPAD // _RB


def _tc_pre_body(h_ref, deg_ref, o_ref):
    norm = lax.rsqrt(jnp.maximum(deg_ref[:, :1], 1.0))
    o_ref[...] = h_ref[...] * norm


def _tc_pre(h_pad, out_deg):
    return pl.pallas_call(
        _tc_pre_body,
        grid=(_GRID,),
        in_specs=[pl.BlockSpec((_RB, D), lambda i: (i, 0)),
                  pl.BlockSpec((_RB, D), lambda i: (i, 0))],
        out_specs=pl.BlockSpec((_RB, D), lambda i: (i, 0)),
        out_shape=jax.ShapeDtypeStruct((NPAD, D), jnp.float32),
    )(h_pad, out_deg)


def _tc_mid_body(a_ref, ind_ref, outd_ref, w_ref, bias_ref, o_ref):
    agg = a_ref[...] * lax.rsqrt(jnp.maximum(ind_ref[:, :1], 1.0))
    y = jnp.dot(agg, w_ref[...], preferred_element_type=jnp.float32,
                precision=lax.Precision.HIGHEST)
    y = jnp.maximum(y + bias_ref[...], 0.0)
    o_ref[...] = y * lax.rsqrt(jnp.maximum(outd_ref[:, :1], 1.0))


def _tc_mid(acc, in_deg, out_deg, W, bias):
    return pl.pallas_call(
        _tc_mid_body,
        grid=(_GRID,),
        in_specs=[pl.BlockSpec((_RB, D), lambda i: (i, 0)),
                  pl.BlockSpec((_RB, D), lambda i: (i, 0)),
                  pl.BlockSpec((_RB, D), lambda i: (i, 0)),
                  pl.BlockSpec((D, D), lambda i: (0, 0)),
                  pl.BlockSpec((1, D), lambda i: (0, 0))],
        out_specs=pl.BlockSpec((_RB, D), lambda i: (i, 0)),
        out_shape=jax.ShapeDtypeStruct((NPAD, D), jnp.float32),
    )(acc, in_deg, out_deg, W, bias)


def _tc_last_body(a_ref, ind_ref, w_ref, bias_ref, o_ref):
    agg = a_ref[...] * lax.rsqrt(jnp.maximum(ind_ref[:, :1], 1.0))
    o_ref[...] = jnp.dot(agg, w_ref[...], preferred_element_type=jnp.float32,
                         precision=lax.Precision.HIGHEST) + bias_ref[...]


def _tc_last(acc, in_deg, W, bias):
    return pl.pallas_call(
        _tc_last_body,
        grid=(_GRID,),
        in_specs=[pl.BlockSpec((_RB, D), lambda i: (i, 0)),
                  pl.BlockSpec((_RB, D), lambda i: (i, 0)),
                  pl.BlockSpec((D, D), lambda i: (0, 0)),
                  pl.BlockSpec((1, D), lambda i: (0, 0))],
        out_specs=pl.BlockSpec((_RB, D), lambda i: (i, 0)),
        out_shape=jax.ShapeDtypeStruct((N, D), jnp.float32),
    )(acc, in_deg, W, bias)


def kernel(h, e, edge_index, W0, b0, W1, b1, W2, b2):
    src = edge_index[0]
    dst = edge_index[1]
    pad_idx = jnp.full((EPAD - E,), PADROW, dtype=jnp.int32)
    src_t = jnp.concatenate([src, pad_idx]).reshape(16, NCH, CH)
    dst_t = jnp.concatenate([dst, pad_idx]).reshape(16, NCH, CH)
    h_pad = jnp.pad(h, ((0, NPAD - N), (0, 0)))
    ones2 = jnp.ones((NPAD, D), jnp.float32)
    zeroacc = jnp.zeros((ACCR, D), jnp.float32)

    out_deg = _sc_pass(ones2, src_t, src_t, zeroacc)
    in_deg = _sc_pass(ones2, src_t, dst_t, zeroacc)
    h2 = _tc_pre(h_pad, out_deg)

    out = None
    for (W, b, last) in ((W0, b0, False), (W1, b1, False), (W2, b2, True)):
        agg = _sc_pass(h2, src_t, dst_t, zeroacc)
        if last:
            out = _tc_last(agg, in_deg, W, b.reshape(1, D))
        else:
            h2 = _tc_mid(agg, in_deg, out_deg, W, b.reshape(1, D))
    return (out, e)


# speedup vs baseline: 1.3703x; 1.0691x over previous
import functools

import jax
import jax.numpy as jnp
from jax import lax
from jax.experimental import pallas as pl
from jax.experimental.pallas import tpu as pltpu
from jax.experimental.pallas import tpu_sc as plsc

N = 10000
E = 320000
D = 128
NH = 5120
NPAD = 2 * NH
ACCR = 5248
TRASH = 5184
ZRPT = ACCR // 16
ORPT = NH // 16
CH = 128
NCH = 160
EPT = NCH * CH
EPAD = 16 * EPT
PADROW = N

_MESH = plsc.VectorSubcoreMesh(core_axis_name="c", subcore_axis_name="s")


@functools.partial(
    pl.kernel,
    out_type=jax.ShapeDtypeStruct((NPAD, D), jnp.float32),
    mesh=_MESH,
    scratch_types=[
        pltpu.VMEM_SHARED((ACCR, D), jnp.float32),
        pltpu.VMEM((NCH, CH), jnp.int32),
        pltpu.VMEM((NCH, CH), jnp.int32),
        pltpu.VMEM((2, CH), jnp.int32),
        pltpu.VMEM((2, CH), jnp.int32),
        pltpu.VMEM((CH, D), jnp.float32),
        pltpu.VMEM((CH, D), jnp.float32),
        pltpu.SemaphoreType.DMA,
        pltpu.SemaphoreType.DMA,
        pltpu.SemaphoreType.DMA,
        pltpu.SemaphoreType.DMA,
    ],
)
def _sc_pass(table_h, si, di, zeros, out,
             acc, sidx, didx, sidx1, didx1, rows0, rows1,
             gsem0, gsem1, ssem0, ssem1):
    c = lax.axis_index("c")
    s = lax.axis_index("s")
    base = c * NH

    pltpu.sync_copy(zeros.at[pl.ds(s * ZRPT, ZRPT)],
                    acc.at[pl.ds(s * ZRPT, ZRPT)])
    pltpu.sync_copy(si.at[s], sidx)
    pltpu.sync_copy(di.at[s], didx)
    plsc.subcore_barrier()

    rows = (rows0, rows1)
    gsem = (gsem0, gsem1)
    ssem = (ssem0, ssem1)

    def fill_sidx(p, j):
        for k in range(CH // 16):
            sl = pl.ds(k * 16, 16)
            sidx1[p, sl] = sidx[j, sl]

    def fill_didx(p, j):
        for k in range(CH // 16):
            sl = pl.ds(k * 16, 16)
            v = didx[j, sl] - base
            ok = (v >= 0) & (v < NH)
            didx1[p, sl] = jnp.where(ok, v, TRASH)

    def gather(p, j):
        del j
        return pltpu.async_copy(table_h.at[sidx1.at[p]], rows[p], gsem[p])

    def scatter(p):
        return pltpu.async_copy(rows[p], acc.at[didx1.at[p]], ssem[p],
                                add=True)

    for p in (0, 1):
        fill_sidx(p, p)
        fill_didx(p, p)
        gather(p, p)

    @pl.loop(0, NCH // 2 - 1)
    def _(jj):
        for p in (0, 1):
            j = 2 * jj + p
            pltpu.make_async_copy(table_h, rows[p], gsem[p]).wait()
            d = scatter(p)
            fill_sidx(p, j + 2)
            d.wait()
            fill_didx(p, j + 2)
            gather(p, j + 2)

    for p in (0, 1):
        pltpu.make_async_copy(table_h, rows[p], gsem[p]).wait()
        scatter(p).wait()

    plsc.subcore_barrier()
    pltpu.sync_copy(acc.at[pl.ds(s * ORPT, ORPT)],
                    out.at[pl.ds(base + s * ORPT, ORPT)])


_RB = 512
_GRID = NPAD // _RB


def _tc_pre_body(h_ref, deg_ref, o_ref):
    norm = lax.rsqrt(jnp.maximum(deg_ref[:, :1], 1.0))
    o_ref[...] = h_ref[...] * norm


def _tc_pre(h_pad, out_deg):
    return pl.pallas_call(
        _tc_pre_body,
        grid=(_GRID,),
        in_specs=[pl.BlockSpec((_RB, D), lambda i: (i, 0)),
                  pl.BlockSpec((_RB, D), lambda i: (i, 0))],
        out_specs=pl.BlockSpec((_RB, D), lambda i: (i, 0)),
        out_shape=jax.ShapeDtypeStruct((NPAD, D), jnp.float32),
    )(h_pad, out_deg)


def _tc_mid_body(a_ref, ind_ref, outd_ref, w_ref, bias_ref, o_ref):
    agg = a_ref[...] * lax.rsqrt(jnp.maximum(ind_ref[:, :1], 1.0))
    y = jnp.dot(agg, w_ref[...], preferred_element_type=jnp.float32,
                precision=lax.Precision.HIGHEST)
    y = jnp.maximum(y + bias_ref[...], 0.0)
    o_ref[...] = y * lax.rsqrt(jnp.maximum(outd_ref[:, :1], 1.0))


def _tc_mid(acc, in_deg, out_deg, W, bias):
    return pl.pallas_call(
        _tc_mid_body,
        grid=(_GRID,),
        in_specs=[pl.BlockSpec((_RB, D), lambda i: (i, 0)),
                  pl.BlockSpec((_RB, D), lambda i: (i, 0)),
                  pl.BlockSpec((_RB, D), lambda i: (i, 0)),
                  pl.BlockSpec((D, D), lambda i: (0, 0)),
                  pl.BlockSpec((1, D), lambda i: (0, 0))],
        out_specs=pl.BlockSpec((_RB, D), lambda i: (i, 0)),
        out_shape=jax.ShapeDtypeStruct((NPAD, D), jnp.float32),
    )(acc, in_deg, out_deg, W, bias)


def _tc_last_body(a_ref, ind_ref, w_ref, bias_ref, o_ref):
    agg = a_ref[...] * lax.rsqrt(jnp.maximum(ind_ref[:, :1], 1.0))
    o_ref[...] = jnp.dot(agg, w_ref[...], preferred_element_type=jnp.float32,
                         precision=lax.Precision.HIGHEST) + bias_ref[...]


def _tc_last(acc, in_deg, W, bias):
    return pl.pallas_call(
        _tc_last_body,
        grid=(_GRID,),
        in_specs=[pl.BlockSpec((_RB, D), lambda i: (i, 0)),
                  pl.BlockSpec((_RB, D), lambda i: (i, 0)),
                  pl.BlockSpec((D, D), lambda i: (0, 0)),
                  pl.BlockSpec((1, D), lambda i: (0, 0))],
        out_specs=pl.BlockSpec((_RB, D), lambda i: (i, 0)),
        out_shape=jax.ShapeDtypeStruct((N, D), jnp.float32),
    )(acc, in_deg, W, bias)


def kernel(h, e, edge_index, W0, b0, W1, b1, W2, b2):
    src = edge_index[0]
    dst = edge_index[1]
    pad_idx = jnp.full((EPAD - E,), PADROW, dtype=jnp.int32)
    src_t = jnp.concatenate([src, pad_idx]).reshape(16, NCH, CH)
    dst_t = jnp.concatenate([dst, pad_idx]).reshape(16, NCH, CH)
    h_pad = jnp.pad(h, ((0, NPAD - N), (0, 0)))
    ones2 = jnp.ones((NPAD, D), jnp.float32)
    zeroacc = jnp.zeros((ACCR, D), jnp.float32)

    out_deg = _sc_pass(ones2, src_t, src_t, zeroacc)
    in_deg = _sc_pass(ones2, src_t, dst_t, zeroacc)
    h2 = _tc_pre(h_pad, out_deg)

    out = None
    for (W, b, last) in ((W0, b0, False), (W1, b1, False), (W2, b2, True)):
        agg = _sc_pass(h2, src_t, dst_t, zeroacc)
        if last:
            out = _tc_last(agg, in_deg, W, b.reshape(1, D))
        else:
            h2 = _tc_mid(agg, in_deg, out_deg, W, b.reshape(1, D))
    return (out, e)
